# Initial kernel scaffold; baseline (speedup 1.0000x reference)
import functools

import jax
import jax.numpy as jnp
from jax import lax
from jax.experimental import pallas as pl
from jax.experimental.pallas import tpu as pltpu
from jax.experimental.pallas import tpu_sc as plsc

N = 10000
E = 320000
NPG = 80
NB = 125
LC = 128
LC4 = 32
LD = 256

NCORE, NSUB = 2, 16
NW = NCORE * NSUB
NROW = 10016
SLAB = NROW // NSUB
HSLAB = SLAB // 2
EPAD = 327680
IDXROWS = EPAD // 128
WROWS = IDXROWS // NW
GROUPS = WROWS // 4

RB = 2000
GRID = N // RB

_sc_mesh = plsc.VectorSubcoreMesh(core_axis_name="c", subcore_axis_name="s")


def _dot(a, b):
    return jnp.dot(a, b, preferred_element_type=jnp.float32)


@functools.partial(
    pl.kernel,
    out_type=jax.ShapeDtypeStruct((NCORE, NROW, LC), jnp.float32),
    mesh=_sc_mesh,
    scratch_types=[
        pltpu.VMEM((4, 128), jnp.int32),
        pltpu.VMEM((4, 128), jnp.int32),
        pltpu.VMEM((4, 128, LC), jnp.float32),
        pltpu.VMEM((HSLAB, LC), jnp.float32),
        pltpu.VMEM_SHARED((NROW, LC), jnp.float32),
        pltpu.SemaphoreType.DMA,
    ],
)
def _sc_segsum(bm, srcg, dstg, out, sidx, didx, rows, zbuf, acc, sem):
    c = lax.axis_index("c")
    s = lax.axis_index("s")
    w = s * NCORE + c

    def zrow(i, carry):
        for j in range(LC // 16):
            zbuf[i, pl.ds(j * 16, 16)] = jnp.zeros((16,), jnp.float32)
        return carry

    lax.fori_loop(0, HSLAB, zrow, 0)
    pltpu.sync_copy(zbuf, acc.at[pl.ds(s * SLAB, HSLAB)])
    pltpu.sync_copy(zbuf, acc.at[pl.ds(s * SLAB + HSLAB, HSLAB)])
    plsc.subcore_barrier()

    def group(g, carry):
        base = w * WROWS + g * 4
        pltpu.sync_copy(srcg.at[pl.ds(base, 4)], sidx)
        pltpu.sync_copy(dstg.at[pl.ds(base, 4)], didx)
        cps = [pltpu.async_copy(bm.at[sidx.at[j]], rows.at[j], sem)
               for j in range(4)]
        for cp in cps:
            cp.wait()
        for j in range(4):
            pltpu.sync_copy(rows.at[j], acc.at[didx.at[j]], add=True)
        return carry

    lax.fori_loop(0, GROUPS, group, 0)
    plsc.subcore_barrier()
    pltpu.sync_copy(acc.at[pl.ds(s * SLAB, SLAB)],
                    out.at[c, pl.ds(s * SLAB, SLAB)])


@functools.partial(
    pl.kernel,
    out_type=jax.ShapeDtypeStruct((NCORE, NROW, 16), jnp.float32),
    mesh=_sc_mesh,
    scratch_types=[
        pltpu.VMEM((4, 128), jnp.int32),
        pltpu.VMEM((128, 16), jnp.float32),
        pltpu.VMEM((SLAB, 16), jnp.float32),
        pltpu.VMEM_SHARED((NROW, 16), jnp.float32),
    ],
)
def _sc_deg(dstg, out, didx, ones, zbuf, acc):
    c = lax.axis_index("c")
    s = lax.axis_index("s")
    w = s * NCORE + c

    def orow(i, carry):
        ones[i, :] = jnp.ones((16,), jnp.float32)
        return carry

    def zrow(i, carry):
        zbuf[i, :] = jnp.zeros((16,), jnp.float32)
        return carry

    lax.fori_loop(0, 128, orow, 0)
    lax.fori_loop(0, SLAB, zrow, 0)
    pltpu.sync_copy(zbuf, acc.at[pl.ds(s * SLAB, SLAB)])
    plsc.subcore_barrier()

    def group(g, carry):
        base = w * WROWS + g * 4
        pltpu.sync_copy(dstg.at[pl.ds(base, 4)], didx)
        for j in range(4):
            pltpu.sync_copy(ones, acc.at[didx.at[j]], add=True)
        return carry

    lax.fori_loop(0, GROUPS, group, 0)
    plsc.subcore_barrier()
    pltpu.sync_copy(acc.at[pl.ds(s * SLAB, SLAB)],
                    out.at[c, pl.ds(s * SLAB, SLAB)])


def _init_body(bsh, biou, osz, opos, x, ohft,
               Wsh, bshb, Wio, bio, Wsz, bsz, Wp, bp, Wx, bx, Wft, bft,
               W1d, W1s, n0_o, a1_o, b1_o):
    bsf = _dot(bsh[...], Wsh[...]) + bshb[...]
    bif = biou[...] * Wio[...][0:1, :] + bio[...]
    size = jnp.maximum(_dot(osz[...], Wsz[...]) + bsz[...], 0.0)
    pos = jnp.maximum(_dot(opos[...], Wp[...]) + bp[...], 0.0)
    xe = _dot(x[...], Wx[...]) + bx[...]
    ft = jnp.maximum(_dot(xe, Wft[...]) + ohft[...] + bft[...], 0.0)
    n0 = jnp.concatenate([bsf, bif, size, pos, ft], axis=1)
    n0_o[...] = n0
    a1_o[...] = _dot(n0, W1d[...])
    b1_o[...] = _dot(n0, W1s[...])


def _scale_body(degp, sc_o):
    deg = degp[...][0, :, 0:1] + degp[...][1, :, 0:1]
    scl = jnp.where(deg > 0.0, 1.0 / jnp.maximum(deg, 1.0), 0.0)
    sc_o[...] = jnp.broadcast_to(scl, (NROW, 8))


def _combine_body(a, sp, sc, bias, wnd, wns, n_o, an_o, bn_o):
    S = sp[...][0] + sp[...][1]
    scl = sc[...][:, 0:1]
    mask = (scl > 0.0).astype(jnp.float32)
    n = jnp.maximum((a[...] + bias[...]) * mask + S * scl, 0.0)
    n_o[...] = n
    an_o[...] = _dot(n, wnd[...])
    bn_o[...] = _dot(n, wns[...])


def _combine_last_body(a, sp, sc, bias, n_o):
    S = sp[...][0] + sp[...][1]
    scl = sc[...][:, 0:1]
    mask = (scl > 0.0).astype(jnp.float32)
    n_o[...] = jnp.maximum((a[...] + bias[...]) * mask + S * scl, 0.0)


def _pool_body(n0, n1, n2, n3, P, g_o):
    cat = jnp.concatenate([n0[...], n1[...], n2[...], n3[...]], axis=1)
    g_o[...] = _dot(P[...], cat)


def _latent_body(g, bl, bsc, eps,
                 Wb0, bb0, Wb1, bb1, Wscw, bscb, Wagg, bagg,
                 Wmu, bmu, Wvar, bvar, mu_o, lv_o, z_o):
    obs = _dot(jnp.maximum(_dot(bl[...], Wb0[...]) + bb0[...], 0.0),
               Wb1[...]) + bb1[...]
    osc = bsc[...] * Wscw[...][0:1, :] + bscb[...]
    gcat = jnp.concatenate([g[...], obs, osc], axis=1)
    latent = _dot(gcat, Wagg[...]) + bagg[...]
    mu = _dot(latent, Wmu[...]) + bmu[...]
    lv = _dot(latent, Wvar[...]) + bvar[...]
    mu_o[...] = mu
    lv_o[...] = lv
    z_o[...] = eps[...] * jnp.exp(0.5 * lv) + mu


def _dft_body(z, Wdf, bdf, zz_o):
    zz_o[...] = _dot(z[...], Wdf[...]) + bdf[...]


def _dpre_body(zz, oha, ohb, Wa, Wb, a_o, b_o):
    r = jnp.maximum(zz[...], 0.0)
    a_o[...] = _dot(r, Wa[...]) + oha[...]
    b_o[...] = _dot(r, Wb[...]) + ohb[...]


def _heads_body(d3, Wcat, bcat, W2, b2c, We, o_o):
    h = jnp.maximum(_dot(d3[...], Wcat[...]) + bcat[...], 0.0)
    o_o[...] = _dot(h, W2[...]) + b2c[...] + _dot(d3[...], We[...])


def _full(x):
    nd = x.ndim
    return pl.BlockSpec(x.shape, lambda i: (0,) * nd)


def _rows(shape):
    nz = len(shape) - 1
    return pl.BlockSpec(shape, lambda i: (i,) + (0,) * nz)


def kernel(x, edge_index, node_pos, node_size, org_node_pos, org_node_size,
           b_shape, b_iou, asp_rto_gt, long_side_gt, blockshape_latent_gt,
           block_scale_gt, ptr, batch, eps, params):
    p = params
    f32 = jnp.float32

    src = edge_index[0]
    dst = edge_index[1]
    srcg = jnp.concatenate(
        [src, jnp.zeros((EPAD - E,), jnp.int32)]).reshape(IDXROWS, 128)
    dstg = jnp.concatenate(
        [dst, jnp.full((EPAD - E,), N, jnp.int32)]).reshape(IDXROWS, 128)

    W1 = p["e_conv1"]["W"]
    W2 = p["e_conv2"]["W"]
    W3 = p["e_conv3"]["W"]
    Wd1 = p["d_conv1"]["W"]
    Wd2 = p["d_conv2"]["W"]
    Wd3 = p["d_conv3"]["W"]
    ohft = jnp.tile(p["ft_init"]["W"][LC4:], (RB // NPG, 1))
    oha = jnp.tile(Wd1[LC:LC + NPG], (RB // NPG, 1))
    ohb = jnp.tile(Wd1[2 * LC + NPG:], (RB // NPG, 1))
    P = jnp.repeat(jnp.eye(RB // NPG, dtype=f32), NPG, axis=1) / NPG

    names = ["d_posx", "d_posy", "d_sizex", "d_sizey", "d_shape", "d_iou"]
    outdims = [1, 1, 1, 1, 6, 1]
    Wcat = jnp.concatenate([p[nm + "_0"]["W"] for nm in names], axis=1)
    bcat = jnp.concatenate([p[nm + "_0"]["b"] for nm in names])[None]
    cols = []
    for i, (nm, od) in enumerate(zip(names, outdims)):
        blk = jnp.zeros((LC * 6, od), f32).at[i * LC:(i + 1) * LC].set(
            p[nm + "_1"]["W"])
        cols.append(blk)
    W2blk = jnp.concatenate(cols + [jnp.zeros((LC * 6, 5), f32)], axis=1)
    b2cat = jnp.concatenate(
        [p[nm + "_1"]["b"] for nm in names]
        + [p["d_exist_1"]["b"], jnp.zeros((4,), f32)])[None]
    We16 = jnp.zeros((LC, 16), f32).at[:, 11:12].set(p["d_exist_1"]["W"])

    def b2(v):
        return v[None]

    degp = _sc_deg(dstg)
    scale = pl.pallas_call(
        _scale_body,
        grid=(1,),
        in_specs=[pl.BlockSpec((NCORE, NROW, 16), lambda i: (0, 0, 0))],
        out_specs=pl.BlockSpec((NROW, 8), lambda i: (0, 0)),
        out_shape=jax.ShapeDtypeStruct((NROW, 8), f32),
    )(degp)

    n0, A1, B1 = pl.pallas_call(
        _init_body,
        grid=(GRID,),
        in_specs=[
            _rows((RB, 6)), _rows((RB, 1)), _rows((RB, 2)), _rows((RB, 2)),
            _rows((RB, 2)), _rows((RB, 64)),
            _full(p["enc_shape"]["W"]), _full(b2(p["enc_shape"]["b"])),
            _full(p["enc_iou"]["W"]), _full(b2(p["enc_iou"]["b"])),
            _full(p["size_init"]["W"]), _full(b2(p["size_init"]["b"])),
            _full(p["pos_init"]["W"]), _full(b2(p["pos_init"]["b"])),
            _full(p["ex_init"]["W"]), _full(b2(p["ex_init"]["b"])),
            _full(p["ft_init"]["W"][:LC4]), _full(b2(p["ft_init"]["b"])),
            _full(W1[:2 * LC]), _full(W1[2 * LC:]),
        ],
        out_specs=[_rows((RB, 256)), _rows((RB, 128)), _rows((RB, 128))],
        out_shape=[jax.ShapeDtypeStruct((N, 256), f32),
                   jax.ShapeDtypeStruct((N, 128), f32),
                   jax.ShapeDtypeStruct((N, 128), f32)],
    )(b_shape, b_iou, org_node_size, org_node_pos, x, ohft,
      p["enc_shape"]["W"], b2(p["enc_shape"]["b"]),
      p["enc_iou"]["W"], b2(p["enc_iou"]["b"]),
      p["size_init"]["W"], b2(p["size_init"]["b"]),
      p["pos_init"]["W"], b2(p["pos_init"]["b"]),
      p["ex_init"]["W"], b2(p["ex_init"]["b"]),
      p["ft_init"]["W"][:LC4], b2(p["ft_init"]["b"]),
      W1[:2 * LC], W1[2 * LC:])

    def combine(A, Sp, bias, wnd=None, wns=None):
        if wnd is None:
            return pl.pallas_call(
                _combine_last_body,
                grid=(GRID,),
                in_specs=[
                    _rows((RB, 128)),
                    pl.BlockSpec((NCORE, RB, LC), lambda i: (0, i, 0)),
                    _rows((RB, 8)), _full(bias),
                ],
                out_specs=_rows((RB, 128)),
                out_shape=jax.ShapeDtypeStruct((N, 128), f32),
            )(A, Sp, scale, bias)
        return pl.pallas_call(
            _combine_body,
            grid=(GRID,),
            in_specs=[
                _rows((RB, 128)),
                pl.BlockSpec((NCORE, RB, LC), lambda i: (0, i, 0)),
                _rows((RB, 8)), _full(bias), _full(wnd), _full(wns),
            ],
            out_specs=[_rows((RB, 128))] * 3,
            out_shape=[jax.ShapeDtypeStruct((N, 128), f32)] * 3,
        )(A, Sp, scale, bias, wnd, wns)

    S1 = _sc_segsum(B1, srcg, dstg)
    n1, A2, B2 = combine(A1, S1, b2(p["e_conv1"]["b"]), W2[:LC], W2[LC:])
    S2 = _sc_segsum(B2, srcg, dstg)
    n2, A3, B3 = combine(A2, S2, b2(p["e_conv2"]["b"]), W3[:LC], W3[LC:])
    S3 = _sc_segsum(B3, srcg, dstg)
    n3 = combine(A3, S3, b2(p["e_conv3"]["b"]))

    g = pl.pallas_call(
        _pool_body,
        grid=(GRID,),
        in_specs=[_rows((RB, 256))] + [_rows((RB, 128))] * 3 + [_full(P)],
        out_specs=_rows((RB // NPG, 640)),
        out_shape=jax.ShapeDtypeStruct((NB, 640), f32),
    )(n0, n1, n2, n3, P)

    g128 = jnp.zeros((128, 640), f32).at[:NB].set(g)
    bl128 = jnp.zeros((128, 32), f32).at[:NB].set(blockshape_latent_gt)
    bsc128 = jnp.zeros((128, 1), f32).at[:NB, 0].set(block_scale_gt)
    eps128 = jnp.zeros((128, LD), f32).at[:NB].set(eps)

    mu128, lv128, z128 = pl.pallas_call(
        _latent_body,
        grid=(1,),
        in_specs=[_full(g128), _full(bl128), _full(bsc128), _full(eps128),
                  _full(p["enc_block_shape_0"]["W"]),
                  _full(b2(p["enc_block_shape_0"]["b"])),
                  _full(p["enc_block_shape_1"]["W"]),
                  _full(b2(p["enc_block_shape_1"]["b"])),
                  _full(p["enc_block_scale"]["W"]),
                  _full(b2(p["enc_block_scale"]["b"])),
                  _full(p["aggregate"]["W"]), _full(b2(p["aggregate"]["b"])),
                  _full(p["fc_mu"]["W"]), _full(b2(p["fc_mu"]["b"])),
                  _full(p["fc_var"]["W"]), _full(b2(p["fc_var"]["b"]))],
        out_specs=[pl.BlockSpec((128, LD), lambda i: (0, 0))] * 3,
        out_shape=[jax.ShapeDtypeStruct((128, LD), f32)] * 3,
    )(g128, bl128, bsc128, eps128,
      p["enc_block_shape_0"]["W"], b2(p["enc_block_shape_0"]["b"]),
      p["enc_block_shape_1"]["W"], b2(p["enc_block_shape_1"]["b"]),
      p["enc_block_scale"]["W"], b2(p["enc_block_scale"]["b"]),
      p["aggregate"]["W"], b2(p["aggregate"]["b"]),
      p["fc_mu"]["W"], b2(p["fc_mu"]["b"]),
      p["fc_var"]["W"], b2(p["fc_var"]["b"]))

    mu = mu128[:NB]
    log_var = lv128[:NB]

    DB = 1280
    zzfull = pl.pallas_call(
        _dft_body,
        grid=(10240 // DB,),
        in_specs=[
            pl.BlockSpec((128, LD), lambda i: (0, 0)),
            pl.BlockSpec((LD, DB), lambda i: (0, i)),
            pl.BlockSpec((1, DB), lambda i: (0, i)),
        ],
        out_specs=pl.BlockSpec((128, DB), lambda i: (0, i)),
        out_shape=jax.ShapeDtypeStruct((128, 10240), f32),
    )(z128, p["d_ft_init"]["W"], b2(p["d_ft_init"]["b"]))
    zz = zzfull[:NB].reshape(N, LC)

    Ad1, Bd1 = pl.pallas_call(
        _dpre_body,
        grid=(GRID,),
        in_specs=[_rows((RB, 128)), _rows((RB, 128)), _rows((RB, 128)),
                  _full(Wd1[:LC]), _full(Wd1[LC + NPG:2 * LC + NPG])],
        out_specs=[_rows((RB, 128))] * 2,
        out_shape=[jax.ShapeDtypeStruct((N, 128), f32)] * 2,
    )(zz, oha, ohb, Wd1[:LC], Wd1[LC + NPG:2 * LC + NPG])

    Sd1 = _sc_segsum(Bd1, srcg, dstg)
    d1, Ad2, Bd2 = combine(Ad1, Sd1, b2(p["d_conv1"]["b"]), Wd2[:LC], Wd2[LC:])
    Sd2 = _sc_segsum(Bd2, srcg, dstg)
    d2, Ad3, Bd3 = combine(Ad2, Sd2, b2(p["d_conv2"]["b"]), Wd3[:LC], Wd3[LC:])
    Sd3 = _sc_segsum(Bd3, srcg, dstg)
    d3 = combine(Ad3, Sd3, b2(p["d_conv3"]["b"]))

    o16 = pl.pallas_call(
        _heads_body,
        grid=(GRID,),
        in_specs=[_rows((RB, 128)), _full(Wcat), _full(bcat),
                  _full(W2blk), _full(b2cat), _full(We16)],
        out_specs=_rows((RB, 16)),
        out_shape=jax.ShapeDtypeStruct((N, 16), f32),
    )(d3, Wcat, bcat, W2blk, b2cat, We16)

    exist = o16[:, 11:12]
    pos_o = o16[:, 0:2]
    size_o = o16[:, 2:4]
    b_shape_o = o16[:, 4:10]
    b_iou_o = o16[:, 10:11]
    return (exist, pos_o, size_o, mu, log_var, b_shape_o, b_iou_o)


# speedup vs baseline: 3.4988x; 3.4988x over previous
import functools

import jax
import jax.numpy as jnp
from jax import lax
from jax.experimental import pallas as pl
from jax.experimental.pallas import tpu as pltpu
from jax.experimental.pallas import tpu_sc as plsc

N = 10000
E = 320000
NPG = 80
NB = 125
LC = 128
LC4 = 32
LD = 256

NCORE, NSUB = 2, 16
NW = NCORE * NSUB
NROW = 10112
SLAB = NROW // NSUB
HSLAB = SLAB // 2
EPAD = 327680
IDXROWS = EPAD // 128
WROWS = IDXROWS // NW
DESC = 2
GROUPS = WROWS // DESC
ZROWS = 79

RB = 2000
GRID = N // RB

_sc_mesh = plsc.VectorSubcoreMesh(
    core_axis_name="c", subcore_axis_name="s",
    num_cores=NCORE, num_subcores=NSUB)


def _dot(a, b):
    return jnp.dot(a, b, preferred_element_type=jnp.float32)


def _dot_hi(a, b):
    return jnp.dot(a, b, preferred_element_type=jnp.float32,
                   precision=lax.Precision.HIGHEST)


def _bf16r(x):
    return x.astype(jnp.bfloat16).astype(jnp.float32)


@functools.partial(
    pl.kernel,
    out_type=jax.ShapeDtypeStruct((NCORE, NROW, LC), jnp.float32),
    mesh=_sc_mesh,
    scratch_types=[
        pltpu.VMEM((DESC, 128), jnp.int32),
        pltpu.VMEM((DESC, 128), jnp.int32),
        pltpu.VMEM((DESC, 128, LC), jnp.float32),
        pltpu.VMEM((ZROWS, LC), jnp.float32),
        pltpu.VMEM_SHARED((NROW, LC), jnp.float32),
        pltpu.SemaphoreType.DMA,
    ],
)
def _sc_segsum(bm, srcg, dstg, out, sidx, didx, rows, zbuf, acc, sem):
    c = lax.axis_index("c")
    s = lax.axis_index("s")
    w = s * NCORE + c

    def zrow(i, carry):
        for j in range(LC // 16):
            zbuf[i, pl.ds(j * 16, 16)] = jnp.zeros((16,), jnp.float32)
        return carry

    lax.fori_loop(0, ZROWS, zrow, 0)
    for r in range(SLAB // ZROWS):
        pltpu.sync_copy(zbuf, acc.at[pl.ds(s * SLAB + r * ZROWS, ZROWS)])
    plsc.subcore_barrier()

    def group(g, carry):
        base = w * WROWS + g * DESC
        pltpu.sync_copy(srcg.at[pl.ds(base, DESC)], sidx)
        pltpu.sync_copy(dstg.at[pl.ds(base, DESC)], didx)
        cps = [pltpu.async_copy(bm.at[sidx.at[j]], rows.at[j], sem)
               for j in range(DESC)]
        for cp in cps:
            cp.wait()
        for j in range(DESC):
            pltpu.sync_copy(rows.at[j], acc.at[didx.at[j]], add=True)
        return carry

    lax.fori_loop(0, GROUPS, group, 0)
    plsc.subcore_barrier()
    pltpu.sync_copy(acc.at[pl.ds(s * SLAB, SLAB)],
                    out.at[c, pl.ds(s * SLAB, SLAB)])


@functools.partial(
    pl.kernel,
    out_type=jax.ShapeDtypeStruct((NCORE, NROW, LC), jnp.float32),
    mesh=_sc_mesh,
    scratch_types=[
        pltpu.VMEM((DESC, 128), jnp.int32),
        pltpu.VMEM((128, LC), jnp.float32),
        pltpu.VMEM((ZROWS, LC), jnp.float32),
        pltpu.VMEM_SHARED((NROW, LC), jnp.float32),
    ],
)
def _sc_deg(dstg, out, didx, ones, zbuf, acc):
    c = lax.axis_index("c")
    s = lax.axis_index("s")
    w = s * NCORE + c

    def orow(i, carry):
        for j in range(LC // 16):
            ones[i, pl.ds(j * 16, 16)] = jnp.ones((16,), jnp.float32)
        return carry

    def zrow(i, carry):
        for j in range(LC // 16):
            zbuf[i, pl.ds(j * 16, 16)] = jnp.zeros((16,), jnp.float32)
        return carry

    lax.fori_loop(0, 128, orow, 0)
    lax.fori_loop(0, ZROWS, zrow, 0)
    for r in range(SLAB // ZROWS):
        pltpu.sync_copy(zbuf, acc.at[pl.ds(s * SLAB + r * ZROWS, ZROWS)])
    plsc.subcore_barrier()

    def group(g, carry):
        base = w * WROWS + g * DESC
        pltpu.sync_copy(dstg.at[pl.ds(base, DESC)], didx)
        for j in range(DESC):
            pltpu.sync_copy(ones, acc.at[didx.at[j]], add=True)
        return carry

    lax.fori_loop(0, GROUPS, group, 0)
    plsc.subcore_barrier()
    pltpu.sync_copy(acc.at[pl.ds(s * SLAB, SLAB)],
                    out.at[c, pl.ds(s * SLAB, SLAB)])


def _init_body(bsh, biou, osz, opos, x, ohft,
               Wsh, bshb, Wio, bio, Wsz, bsz, Wp, bp, Wx, bx, Wft, bft,
               W1d, W1s, n0_o, a1_o, b1_o):
    bsf = _dot(bsh[...], Wsh[...]) + bshb[...]
    bif = _bf16r(biou[...]) * Wio[...][0:1, :] + bio[...]
    size = jnp.maximum(_dot(osz[...], Wsz[...]) + bsz[...], 0.0)
    pos = jnp.maximum(_dot(opos[...], Wp[...]) + bp[...], 0.0)
    xe = _dot(x[...], Wx[...]) + bx[...]
    ft = jnp.maximum(_dot(xe, Wft[...]) + ohft[...] + bft[...], 0.0)
    n0 = jnp.concatenate([bsf, bif, size, pos, ft], axis=1)
    n0_o[...] = n0
    a1_o[...] = _dot(n0, W1d[...])
    b1_o[...] = _dot(n0, W1s[...])


def _scale_body(degp, sc_o):
    deg = degp[...][0, :, 0:1] + degp[...][1, :, 0:1]
    scl = jnp.where(deg > 0.0, 1.0 / jnp.maximum(deg, 1.0), 0.0)
    sc_o[...] = jnp.broadcast_to(scl, (NROW, 8))


def _combine_body(a, sp, sc, bias, wnd, wns, n_o, an_o, bn_o):
    S = sp[...][0] + sp[...][1]
    scl = sc[...][:, 0:1]
    mask = (scl > 0.0).astype(jnp.float32)
    n = jnp.maximum((a[...] + bias[...]) * mask + S * scl, 0.0)
    n_o[...] = n
    an_o[...] = _dot(n, wnd[...])
    bn_o[...] = _dot(n, wns[...])


def _combine_last_body(a, sp, sc, bias, n_o):
    S = sp[...][0] + sp[...][1]
    scl = sc[...][:, 0:1]
    mask = (scl > 0.0).astype(jnp.float32)
    n_o[...] = jnp.maximum((a[...] + bias[...]) * mask + S * scl, 0.0)


def _pool_body(n0, n1, n2, n3, P, g_o):
    cat = jnp.concatenate([n0[...], n1[...], n2[...], n3[...]], axis=1)
    g_o[...] = _dot_hi(P[...], cat) / float(NPG)


def _latent_body(g, bl, bsc, eps,
                 Wb0, bb0, Wb1, bb1, Wscw, bscb, Wagg, bagg,
                 Wmu, bmu, Wvar, bvar, mu_o, lv_o, z_o):
    obs = _dot(jnp.maximum(_dot(bl[...], Wb0[...]) + bb0[...], 0.0),
               Wb1[...]) + bb1[...]
    osc = _bf16r(bsc[...]) * Wscw[...][0:1, :] + bscb[...]
    gcat = jnp.concatenate([g[...], obs, osc], axis=1)
    latent = _dot(gcat, Wagg[...]) + bagg[...]
    mu = _dot(latent, Wmu[...]) + bmu[...]
    lv = _dot(latent, Wvar[...]) + bvar[...]
    mu_o[...] = mu
    lv_o[...] = lv
    z_o[...] = eps[...] * jnp.exp(0.5 * lv) + mu


def _dft_body(z, Wdf, bdf, zz_o):
    zz_o[...] = _dot(z[...], Wdf[...]) + bdf[...]


def _dpre_body(zz, oha, ohb, Wa, Wb, a_o, b_o):
    r = jnp.maximum(zz[...], 0.0)
    a_o[...] = _dot(r, Wa[...]) + oha[...]
    b_o[...] = _dot(r, Wb[...]) + ohb[...]


def _heads_body(d3, Wcat, bcat, W2, b2c, We, o_o):
    h = jnp.maximum(_dot(d3[...], Wcat[...]) + bcat[...], 0.0)
    o_o[...] = _dot(h, W2[...]) + b2c[...] + _dot(d3[...], We[...])


def _full(x):
    nd = x.ndim
    return pl.BlockSpec(x.shape, lambda i: (0,) * nd)


def _rows(shape):
    nz = len(shape) - 1
    return pl.BlockSpec(shape, lambda i: (i,) + (0,) * nz)


def kernel(x, edge_index, node_pos, node_size, org_node_pos, org_node_size,
           b_shape, b_iou, asp_rto_gt, long_side_gt, blockshape_latent_gt,
           block_scale_gt, ptr, batch, eps, params):
    p = params
    f32 = jnp.float32

    src = edge_index[0]
    dst = edge_index[1]
    srcg = jnp.concatenate(
        [src, jnp.zeros((EPAD - E,), jnp.int32)]).reshape(IDXROWS, 128)
    dstg = jnp.concatenate(
        [dst, jnp.full((EPAD - E,), N, jnp.int32)]).reshape(IDXROWS, 128)

    W1 = p["e_conv1"]["W"]
    W2 = p["e_conv2"]["W"]
    W3 = p["e_conv3"]["W"]
    Wd1 = p["d_conv1"]["W"]
    Wd2 = p["d_conv2"]["W"]
    Wd3 = p["d_conv3"]["W"]
    ohft = jnp.tile(_bf16r(p["ft_init"]["W"][LC4:]), (RB // NPG, 1))
    oha = jnp.tile(_bf16r(Wd1[LC:LC + NPG]), (RB // NPG, 1))
    ohb = jnp.tile(_bf16r(Wd1[2 * LC + NPG:]), (RB // NPG, 1))
    P = jnp.zeros((32, RB), f32).at[:RB // NPG].set(
        jnp.repeat(jnp.eye(RB // NPG, dtype=f32), NPG, axis=1))

    names = ["d_posx", "d_posy", "d_sizex", "d_sizey", "d_shape", "d_iou"]
    outdims = [1, 1, 1, 1, 6, 1]
    Wcat = jnp.concatenate([p[nm + "_0"]["W"] for nm in names], axis=1)
    bcat = jnp.concatenate([p[nm + "_0"]["b"] for nm in names])[None]
    cols = []
    for i, (nm, od) in enumerate(zip(names, outdims)):
        blk = jnp.zeros((LC * 6, od), f32).at[i * LC:(i + 1) * LC].set(
            p[nm + "_1"]["W"])
        cols.append(blk)
    W2blk = jnp.concatenate(cols + [jnp.zeros((LC * 6, 5), f32)], axis=1)
    b2cat = jnp.concatenate(
        [p[nm + "_1"]["b"] for nm in names]
        + [p["d_exist_1"]["b"], jnp.zeros((4,), f32)])[None]
    We16 = jnp.zeros((LC, 16), f32).at[:, 11:12].set(p["d_exist_1"]["W"])

    def b2(v):
        return v[None]

    degp = _sc_deg(dstg)
    scale = pl.pallas_call(
        _scale_body,
        grid=(1,),
        in_specs=[pl.BlockSpec((NCORE, NROW, LC), lambda i: (0, 0, 0))],
        out_specs=pl.BlockSpec((NROW, 8), lambda i: (0, 0)),
        out_shape=jax.ShapeDtypeStruct((NROW, 8), f32),
    )(degp)

    n0, A1, B1 = pl.pallas_call(
        _init_body,
        grid=(GRID,),
        in_specs=[
            _rows((RB, 6)), _rows((RB, 1)), _rows((RB, 2)), _rows((RB, 2)),
            _rows((RB, 2)), _full(ohft),
            _full(p["enc_shape"]["W"]), _full(b2(p["enc_shape"]["b"])),
            _full(p["enc_iou"]["W"]), _full(b2(p["enc_iou"]["b"])),
            _full(p["size_init"]["W"]), _full(b2(p["size_init"]["b"])),
            _full(p["pos_init"]["W"]), _full(b2(p["pos_init"]["b"])),
            _full(p["ex_init"]["W"]), _full(b2(p["ex_init"]["b"])),
            _full(p["ft_init"]["W"][:LC4]), _full(b2(p["ft_init"]["b"])),
            _full(W1[:2 * LC]), _full(W1[2 * LC:]),
        ],
        out_specs=[_rows((RB, 256)), _rows((RB, 128)), _rows((RB, 128))],
        out_shape=[jax.ShapeDtypeStruct((N, 256), f32),
                   jax.ShapeDtypeStruct((N, 128), f32),
                   jax.ShapeDtypeStruct((N, 128), f32)],
    )(b_shape, b_iou, org_node_size, org_node_pos, x, ohft,
      p["enc_shape"]["W"], b2(p["enc_shape"]["b"]),
      _bf16r(p["enc_iou"]["W"]), b2(p["enc_iou"]["b"]),
      p["size_init"]["W"], b2(p["size_init"]["b"]),
      p["pos_init"]["W"], b2(p["pos_init"]["b"]),
      p["ex_init"]["W"], b2(p["ex_init"]["b"]),
      p["ft_init"]["W"][:LC4], b2(p["ft_init"]["b"]),
      W1[:2 * LC], W1[2 * LC:])

    def combine(A, Sp, bias, wnd=None, wns=None):
        if wnd is None:
            return pl.pallas_call(
                _combine_last_body,
                grid=(GRID,),
                in_specs=[
                    _rows((RB, 128)),
                    pl.BlockSpec((NCORE, RB, LC), lambda i: (0, i, 0)),
                    _rows((RB, 8)), _full(bias),
                ],
                out_specs=_rows((RB, 128)),
                out_shape=jax.ShapeDtypeStruct((N, 128), f32),
            )(A, Sp, scale, bias)
        return pl.pallas_call(
            _combine_body,
            grid=(GRID,),
            in_specs=[
                _rows((RB, 128)),
                pl.BlockSpec((NCORE, RB, LC), lambda i: (0, i, 0)),
                _rows((RB, 8)), _full(bias), _full(wnd), _full(wns),
            ],
            out_specs=[_rows((RB, 128))] * 3,
            out_shape=[jax.ShapeDtypeStruct((N, 128), f32)] * 3,
        )(A, Sp, scale, bias, wnd, wns)

    segsum = _sc_segsum
    S1 = segsum(B1, srcg, dstg)
    n1, A2, B2 = combine(A1, S1, b2(p["e_conv1"]["b"]), W2[:LC], W2[LC:])
    S2 = segsum(B2, srcg, dstg)
    n2, A3, B3 = combine(A2, S2, b2(p["e_conv2"]["b"]), W3[:LC], W3[LC:])
    S3 = segsum(B3, srcg, dstg)
    n3 = combine(A3, S3, b2(p["e_conv3"]["b"]))

    gp = pl.pallas_call(
        _pool_body,
        grid=(GRID,),
        in_specs=[_rows((RB, 256))] + [_rows((RB, 128))] * 3 + [_full(P)],
        out_specs=_rows((32, 640)),
        out_shape=jax.ShapeDtypeStruct((GRID * 32, 640), f32),
    )(n0, n1, n2, n3, P)
    g = gp.reshape(GRID, 32, 640)[:, :RB // NPG].reshape(NB, 640)

    g128 = jnp.zeros((128, 640), f32).at[:NB].set(g)
    bl128 = jnp.zeros((128, 32), f32).at[:NB].set(blockshape_latent_gt)
    bsc128 = jnp.zeros((128, 1), f32).at[:NB, 0].set(block_scale_gt)
    eps128 = jnp.zeros((128, LD), f32).at[:NB].set(eps)

    mu128, lv128, z128 = pl.pallas_call(
        _latent_body,
        grid=(1,),
        in_specs=[_full(g128), _full(bl128), _full(bsc128), _full(eps128),
                  _full(p["enc_block_shape_0"]["W"]),
                  _full(b2(p["enc_block_shape_0"]["b"])),
                  _full(p["enc_block_shape_1"]["W"]),
                  _full(b2(p["enc_block_shape_1"]["b"])),
                  _full(p["enc_block_scale"]["W"]),
                  _full(b2(p["enc_block_scale"]["b"])),
                  _full(p["aggregate"]["W"]), _full(b2(p["aggregate"]["b"])),
                  _full(p["fc_mu"]["W"]), _full(b2(p["fc_mu"]["b"])),
                  _full(p["fc_var"]["W"]), _full(b2(p["fc_var"]["b"]))],
        out_specs=[pl.BlockSpec((128, LD), lambda i: (0, 0))] * 3,
        out_shape=[jax.ShapeDtypeStruct((128, LD), f32)] * 3,
    )(g128, bl128, bsc128, eps128,
      p["enc_block_shape_0"]["W"], b2(p["enc_block_shape_0"]["b"]),
      p["enc_block_shape_1"]["W"], b2(p["enc_block_shape_1"]["b"]),
      _bf16r(p["enc_block_scale"]["W"]), b2(p["enc_block_scale"]["b"]),
      p["aggregate"]["W"], b2(p["aggregate"]["b"]),
      p["fc_mu"]["W"], b2(p["fc_mu"]["b"]),
      p["fc_var"]["W"], b2(p["fc_var"]["b"]))

    mu = mu128[:NB]
    log_var = lv128[:NB]

    DB = 1280
    zzfull = pl.pallas_call(
        _dft_body,
        grid=(10240 // DB,),
        in_specs=[
            pl.BlockSpec((128, LD), lambda i: (0, 0)),
            pl.BlockSpec((LD, DB), lambda i: (0, i)),
            pl.BlockSpec((1, DB), lambda i: (0, i)),
        ],
        out_specs=pl.BlockSpec((128, DB), lambda i: (0, i)),
        out_shape=jax.ShapeDtypeStruct((128, 10240), f32),
    )(z128, p["d_ft_init"]["W"], b2(p["d_ft_init"]["b"]))
    zz = zzfull[:NB].reshape(N, LC)

    Ad1, Bd1 = pl.pallas_call(
        _dpre_body,
        grid=(GRID,),
        in_specs=[_rows((RB, 128)), _full(oha), _full(ohb),
                  _full(Wd1[:LC]), _full(Wd1[LC + NPG:2 * LC + NPG])],
        out_specs=[_rows((RB, 128))] * 2,
        out_shape=[jax.ShapeDtypeStruct((N, 128), f32)] * 2,
    )(zz, oha, ohb, Wd1[:LC], Wd1[LC + NPG:2 * LC + NPG])

    Sd1 = segsum(Bd1, srcg, dstg)
    d1, Ad2, Bd2 = combine(Ad1, Sd1, b2(p["d_conv1"]["b"]), Wd2[:LC], Wd2[LC:])
    Sd2 = segsum(Bd2, srcg, dstg)
    d2, Ad3, Bd3 = combine(Ad2, Sd2, b2(p["d_conv2"]["b"]), Wd3[:LC], Wd3[LC:])
    Sd3 = segsum(Bd3, srcg, dstg)
    d3 = combine(Ad3, Sd3, b2(p["d_conv3"]["b"]))

    o16 = pl.pallas_call(
        _heads_body,
        grid=(GRID,),
        in_specs=[_rows((RB, 128)), _full(Wcat), _full(bcat),
                  _full(W2blk), _full(b2cat), _full(We16)],
        out_specs=_rows((RB, 16)),
        out_shape=jax.ShapeDtypeStruct((N, 16), f32),
    )(d3, Wcat, bcat, W2blk, b2cat, We16)

    exist = o16[:, 11:12]
    pos_o = o16[:, 0:2]
    size_o = o16[:, 2:4]
    b_shape_o = o16[:, 4:10]
    b_iou_o = o16[:, 10:11]
    return (exist, pos_o, size_o, mu, log_var, b_shape_o, b_iou_o)


# speedup vs baseline: 4.6038x; 1.3158x over previous
import functools

import jax
import jax.numpy as jnp
from jax import lax
from jax.experimental import pallas as pl
from jax.experimental.pallas import tpu as pltpu
from jax.experimental.pallas import tpu_sc as plsc

N = 10000
E = 320000
NPG = 80
NB = 125
LC = 128
LC4 = 32
LD = 256

NCORE, NSUB = 2, 16
NW = NCORE * NSUB
NROW = 10112
SLAB = NROW // NSUB
HSLAB = SLAB // 2
EPAD = 327680
IDXROWS = EPAD // 128
WROWS = IDXROWS // NW
DESC = 2
GROUPS = WROWS // DESC
IBLK = 16
NBLK = WROWS // IBLK
ZROWS = 79

RB = 2000
GRID = N // RB

_sc_mesh = plsc.VectorSubcoreMesh(
    core_axis_name="c", subcore_axis_name="s",
    num_cores=NCORE, num_subcores=NSUB)


def _dot(a, b):
    return jnp.dot(a, b, preferred_element_type=jnp.float32)


def _dot_hi(a, b):
    return jnp.dot(a, b, preferred_element_type=jnp.float32,
                   precision=lax.Precision.HIGHEST)


def _bf16r(x):
    return x.astype(jnp.bfloat16).astype(jnp.float32)


@functools.partial(
    pl.kernel,
    out_type=jax.ShapeDtypeStruct((NCORE, NROW, LC), jnp.float32),
    mesh=_sc_mesh,
    scratch_types=[
        pltpu.VMEM((IBLK, 128), jnp.int32),
        pltpu.VMEM((IBLK, 128), jnp.int32),
        pltpu.VMEM((2, 128, LC), jnp.float32),
        pltpu.VMEM((ZROWS, LC), jnp.float32),
        pltpu.VMEM_SHARED((NROW, LC), jnp.float32),
        pltpu.SemaphoreType.DMA,
    ],
)
def _sc_segsum(bm, srcg, dstg, out, sidx, didx, rows, zbuf, acc, sem):
    c = lax.axis_index("c")
    s = lax.axis_index("s")
    w = s * NCORE + c

    def zrow(i, carry):
        for j in range(LC // 16):
            zbuf[i, pl.ds(j * 16, 16)] = jnp.zeros((16,), jnp.float32)
        return carry

    lax.fori_loop(0, ZROWS, zrow, 0)
    for r in range(SLAB // ZROWS):
        pltpu.sync_copy(zbuf, acc.at[pl.ds(s * SLAB + r * ZROWS, ZROWS)])
    plsc.subcore_barrier()

    def block(b, carry):
        base = w * WROWS + b * IBLK
        pltpu.sync_copy(srcg.at[pl.ds(base, IBLK)], sidx)
        pltpu.sync_copy(dstg.at[pl.ds(base, IBLK)], didx)
        cps = [None] * IBLK
        cps[0] = pltpu.async_copy(bm.at[sidx.at[0]], rows.at[0], sem)
        for j in range(IBLK):
            if j + 1 < IBLK:
                cps[j + 1] = pltpu.async_copy(
                    bm.at[sidx.at[j + 1]], rows.at[(j + 1) % 2], sem)
            cps[j].wait()
            pltpu.sync_copy(rows.at[j % 2], acc.at[didx.at[j]], add=True)
        return carry

    lax.fori_loop(0, NBLK, block, 0)
    plsc.subcore_barrier()
    pltpu.sync_copy(acc.at[pl.ds(s * SLAB, SLAB)],
                    out.at[c, pl.ds(s * SLAB, SLAB)])


@functools.partial(
    pl.kernel,
    out_type=jax.ShapeDtypeStruct((NCORE, NROW, LC), jnp.float32),
    mesh=_sc_mesh,
    scratch_types=[
        pltpu.VMEM((IBLK, 128), jnp.int32),
        pltpu.VMEM((128, LC), jnp.float32),
        pltpu.VMEM((ZROWS, LC), jnp.float32),
        pltpu.VMEM_SHARED((NROW, LC), jnp.float32),
    ],
)
def _sc_deg(dstg, out, didx, ones, zbuf, acc):
    c = lax.axis_index("c")
    s = lax.axis_index("s")
    w = s * NCORE + c

    def orow(i, carry):
        for j in range(LC // 16):
            ones[i, pl.ds(j * 16, 16)] = jnp.ones((16,), jnp.float32)
        return carry

    def zrow(i, carry):
        for j in range(LC // 16):
            zbuf[i, pl.ds(j * 16, 16)] = jnp.zeros((16,), jnp.float32)
        return carry

    lax.fori_loop(0, 128, orow, 0)
    lax.fori_loop(0, ZROWS, zrow, 0)
    for r in range(SLAB // ZROWS):
        pltpu.sync_copy(zbuf, acc.at[pl.ds(s * SLAB + r * ZROWS, ZROWS)])
    plsc.subcore_barrier()

    def block(b, carry):
        base = w * WROWS + b * IBLK
        pltpu.sync_copy(dstg.at[pl.ds(base, IBLK)], didx)
        for j in range(IBLK):
            pltpu.sync_copy(ones, acc.at[didx.at[j]], add=True)
        return carry

    lax.fori_loop(0, NBLK, block, 0)
    plsc.subcore_barrier()
    pltpu.sync_copy(acc.at[pl.ds(s * SLAB, SLAB)],
                    out.at[c, pl.ds(s * SLAB, SLAB)])


def _init_body(bsh, biou, osz, opos, x, ohft,
               Wsh, bshb, Wio, bio, Wsz, bsz, Wp, bp, Wx, bx, Wft, bft,
               W1d, W1s, n0_o, a1_o, b1_o):
    bsf = _dot(bsh[...], Wsh[...]) + bshb[...]
    bif = biou[...] * Wio[...][0:1, :] + bio[...]
    size = jnp.maximum(_dot(osz[...], Wsz[...]) + bsz[...], 0.0)
    pos = jnp.maximum(_dot(opos[...], Wp[...]) + bp[...], 0.0)
    xe = _dot(x[...], Wx[...]) + bx[...]
    ft = jnp.maximum(_dot(xe, Wft[...]) + ohft[...] + bft[...], 0.0)
    n0 = jnp.concatenate([bsf, bif, size, pos, ft], axis=1)
    n0_o[...] = n0
    a1_o[...] = _dot(n0, W1d[...])
    b1_o[...] = _dot(n0, W1s[...])


def _scale_body(degp, sc_o):
    deg = degp[...][0, :, 0:1] + degp[...][1, :, 0:1]
    scl = jnp.where(deg > 0.0, 1.0 / jnp.maximum(deg, 1.0), 0.0)
    sc_o[...] = jnp.broadcast_to(scl, (NROW, 8))


def _combine_body(a, sp, sc, bias, wnd, wns, n_o, an_o, bn_o):
    S = sp[...][0] + sp[...][1]
    scl = sc[...][:, 0:1]
    mask = (scl > 0.0).astype(jnp.float32)
    n = jnp.maximum((a[...] + bias[...]) * mask + S * scl, 0.0)
    n_o[...] = n
    an_o[...] = _dot(n, wnd[...])
    bn_o[...] = _dot(n, wns[...])


def _combine_last_body(a, sp, sc, bias, n_o):
    S = sp[...][0] + sp[...][1]
    scl = sc[...][:, 0:1]
    mask = (scl > 0.0).astype(jnp.float32)
    n_o[...] = jnp.maximum((a[...] + bias[...]) * mask + S * scl, 0.0)


def _pool_body(n0, n1, n2, n3, P, g_o):
    cat = jnp.concatenate([n0[...], n1[...], n2[...], n3[...]], axis=1)
    g_o[...] = _dot_hi(P[...], cat) / float(NPG)


def _latent_body(g, bl, bsc, eps,
                 Wb0, bb0, Wb1, bb1, Wscw, bscb, Wagg, bagg,
                 Wmu, bmu, Wvar, bvar, mu_o, lv_o, z_o):
    obs = _dot(jnp.maximum(_dot(bl[...], Wb0[...]) + bb0[...], 0.0),
               Wb1[...]) + bb1[...]
    osc = bsc[...] * Wscw[...][0:1, :] + bscb[...]
    gcat = jnp.concatenate([g[...], obs, osc], axis=1)
    latent = _dot(gcat, Wagg[...]) + bagg[...]
    mu = _dot(latent, Wmu[...]) + bmu[...]
    lv = _dot(latent, Wvar[...]) + bvar[...]
    mu_o[...] = mu
    lv_o[...] = lv
    z_o[...] = eps[...] * jnp.exp(0.5 * lv) + mu


def _dft_body(z, Wdf, bdf, zz_o):
    zz_o[...] = _dot(z[...], Wdf[...]) + bdf[...]


def _dpre_body(zz, oha, ohb, Wa, Wb, a_o, b_o):
    r = jnp.maximum(zz[...], 0.0)
    a_o[...] = _dot(r, Wa[...]) + oha[...]
    b_o[...] = _dot(r, Wb[...]) + ohb[...]


def _heads_body(d3, Wcat, bcat, W2, b2c, We, o_o):
    h = jnp.maximum(_dot(d3[...], Wcat[...]) + bcat[...], 0.0)
    o_o[...] = _dot(h, W2[...]) + b2c[...] + _dot(d3[...], We[...])


def _full(x):
    nd = x.ndim
    return pl.BlockSpec(x.shape, lambda i: (0,) * nd)


def _rows(shape):
    nz = len(shape) - 1
    return pl.BlockSpec(shape, lambda i: (i,) + (0,) * nz)


def kernel(x, edge_index, node_pos, node_size, org_node_pos, org_node_size,
           b_shape, b_iou, asp_rto_gt, long_side_gt, blockshape_latent_gt,
           block_scale_gt, ptr, batch, eps, params):
    p = params
    f32 = jnp.float32

    src = edge_index[0]
    dst = edge_index[1]
    srcg = jnp.concatenate(
        [src, jnp.zeros((EPAD - E,), jnp.int32)]).reshape(IDXROWS, 128)
    dstg = jnp.concatenate(
        [dst, jnp.full((EPAD - E,), N, jnp.int32)]).reshape(IDXROWS, 128)

    W1 = p["e_conv1"]["W"]
    W2 = p["e_conv2"]["W"]
    W3 = p["e_conv3"]["W"]
    Wd1 = p["d_conv1"]["W"]
    Wd2 = p["d_conv2"]["W"]
    Wd3 = p["d_conv3"]["W"]
    ohft = jnp.tile(_bf16r(p["ft_init"]["W"][LC4:]), (RB // NPG, 1))
    oha = jnp.tile(_bf16r(Wd1[LC:LC + NPG]), (RB // NPG, 1))
    ohb = jnp.tile(_bf16r(Wd1[2 * LC + NPG:]), (RB // NPG, 1))
    P = jnp.zeros((32, RB), f32).at[:RB // NPG].set(
        jnp.repeat(jnp.eye(RB // NPG, dtype=f32), NPG, axis=1))

    names = ["d_posx", "d_posy", "d_sizex", "d_sizey", "d_shape", "d_iou"]
    outdims = [1, 1, 1, 1, 6, 1]
    Wcat = jnp.concatenate([p[nm + "_0"]["W"] for nm in names], axis=1)
    bcat = jnp.concatenate([p[nm + "_0"]["b"] for nm in names])[None]
    cols = []
    for i, (nm, od) in enumerate(zip(names, outdims)):
        blk = jnp.zeros((LC * 6, od), f32).at[i * LC:(i + 1) * LC].set(
            p[nm + "_1"]["W"])
        cols.append(blk)
    W2blk = jnp.concatenate(cols + [jnp.zeros((LC * 6, 5), f32)], axis=1)
    b2cat = jnp.concatenate(
        [p[nm + "_1"]["b"] for nm in names]
        + [p["d_exist_1"]["b"], jnp.zeros((4,), f32)])[None]
    We16 = jnp.zeros((LC, 16), f32).at[:, 11:12].set(p["d_exist_1"]["W"])

    def b2(v):
        return v[None]

    degp = _sc_deg(dstg)
    scale = pl.pallas_call(
        _scale_body,
        grid=(1,),
        in_specs=[pl.BlockSpec((NCORE, NROW, LC), lambda i: (0, 0, 0))],
        out_specs=pl.BlockSpec((NROW, 8), lambda i: (0, 0)),
        out_shape=jax.ShapeDtypeStruct((NROW, 8), f32),
    )(degp)

    n0, A1, B1 = pl.pallas_call(
        _init_body,
        grid=(GRID,),
        in_specs=[
            _rows((RB, 6)), _rows((RB, 1)), _rows((RB, 2)), _rows((RB, 2)),
            _rows((RB, 2)), _full(ohft),
            _full(p["enc_shape"]["W"]), _full(b2(p["enc_shape"]["b"])),
            _full(p["enc_iou"]["W"]), _full(b2(p["enc_iou"]["b"])),
            _full(p["size_init"]["W"]), _full(b2(p["size_init"]["b"])),
            _full(p["pos_init"]["W"]), _full(b2(p["pos_init"]["b"])),
            _full(p["ex_init"]["W"]), _full(b2(p["ex_init"]["b"])),
            _full(p["ft_init"]["W"][:LC4]), _full(b2(p["ft_init"]["b"])),
            _full(W1[:2 * LC]), _full(W1[2 * LC:]),
        ],
        out_specs=[_rows((RB, 256)), _rows((RB, 128)), _rows((RB, 128))],
        out_shape=[jax.ShapeDtypeStruct((N, 256), f32),
                   jax.ShapeDtypeStruct((N, 128), f32),
                   jax.ShapeDtypeStruct((N, 128), f32)],
    )(b_shape, b_iou, org_node_size, org_node_pos, x, ohft,
      p["enc_shape"]["W"], b2(p["enc_shape"]["b"]),
      p["enc_iou"]["W"], b2(p["enc_iou"]["b"]),
      p["size_init"]["W"], b2(p["size_init"]["b"]),
      p["pos_init"]["W"], b2(p["pos_init"]["b"]),
      p["ex_init"]["W"], b2(p["ex_init"]["b"]),
      p["ft_init"]["W"][:LC4], b2(p["ft_init"]["b"]),
      W1[:2 * LC], W1[2 * LC:])

    def combine(A, Sp, bias, wnd=None, wns=None):
        if wnd is None:
            return pl.pallas_call(
                _combine_last_body,
                grid=(GRID,),
                in_specs=[
                    _rows((RB, 128)),
                    pl.BlockSpec((NCORE, RB, LC), lambda i: (0, i, 0)),
                    _rows((RB, 8)), _full(bias),
                ],
                out_specs=_rows((RB, 128)),
                out_shape=jax.ShapeDtypeStruct((N, 128), f32),
            )(A, Sp, scale, bias)
        return pl.pallas_call(
            _combine_body,
            grid=(GRID,),
            in_specs=[
                _rows((RB, 128)),
                pl.BlockSpec((NCORE, RB, LC), lambda i: (0, i, 0)),
                _rows((RB, 8)), _full(bias), _full(wnd), _full(wns),
            ],
            out_specs=[_rows((RB, 128))] * 3,
            out_shape=[jax.ShapeDtypeStruct((N, 128), f32)] * 3,
        )(A, Sp, scale, bias, wnd, wns)

    segsum = _sc_segsum
    S1 = segsum(B1, srcg, dstg)
    n1, A2, B2 = combine(A1, S1, b2(p["e_conv1"]["b"]), W2[:LC], W2[LC:])
    S2 = segsum(B2, srcg, dstg)
    n2, A3, B3 = combine(A2, S2, b2(p["e_conv2"]["b"]), W3[:LC], W3[LC:])
    S3 = segsum(B3, srcg, dstg)
    n3 = combine(A3, S3, b2(p["e_conv3"]["b"]))

    gp = pl.pallas_call(
        _pool_body,
        grid=(GRID,),
        in_specs=[_rows((RB, 256))] + [_rows((RB, 128))] * 3 + [_full(P)],
        out_specs=_rows((32, 640)),
        out_shape=jax.ShapeDtypeStruct((GRID * 32, 640), f32),
    )(n0, n1, n2, n3, P)
    g = gp.reshape(GRID, 32, 640)[:, :RB // NPG].reshape(NB, 640)

    g128 = jnp.zeros((128, 640), f32).at[:NB].set(g)
    bl128 = jnp.zeros((128, 32), f32).at[:NB].set(blockshape_latent_gt)
    bsc128 = jnp.zeros((128, 1), f32).at[:NB, 0].set(block_scale_gt)
    eps128 = jnp.zeros((128, LD), f32).at[:NB].set(eps)

    mu128, lv128, z128 = pl.pallas_call(
        _latent_body,
        grid=(1,),
        in_specs=[_full(g128), _full(bl128), _full(bsc128), _full(eps128),
                  _full(p["enc_block_shape_0"]["W"]),
                  _full(b2(p["enc_block_shape_0"]["b"])),
                  _full(p["enc_block_shape_1"]["W"]),
                  _full(b2(p["enc_block_shape_1"]["b"])),
                  _full(p["enc_block_scale"]["W"]),
                  _full(b2(p["enc_block_scale"]["b"])),
                  _full(p["aggregate"]["W"]), _full(b2(p["aggregate"]["b"])),
                  _full(p["fc_mu"]["W"]), _full(b2(p["fc_mu"]["b"])),
                  _full(p["fc_var"]["W"]), _full(b2(p["fc_var"]["b"]))],
        out_specs=[pl.BlockSpec((128, LD), lambda i: (0, 0))] * 3,
        out_shape=[jax.ShapeDtypeStruct((128, LD), f32)] * 3,
    )(g128, bl128, bsc128, eps128,
      p["enc_block_shape_0"]["W"], b2(p["enc_block_shape_0"]["b"]),
      p["enc_block_shape_1"]["W"], b2(p["enc_block_shape_1"]["b"]),
      p["enc_block_scale"]["W"], b2(p["enc_block_scale"]["b"]),
      p["aggregate"]["W"], b2(p["aggregate"]["b"]),
      p["fc_mu"]["W"], b2(p["fc_mu"]["b"]),
      p["fc_var"]["W"], b2(p["fc_var"]["b"]))

    mu = mu128[:NB]
    log_var = lv128[:NB]

    DB = 1280
    zzfull = pl.pallas_call(
        _dft_body,
        grid=(10240 // DB,),
        in_specs=[
            pl.BlockSpec((128, LD), lambda i: (0, 0)),
            pl.BlockSpec((LD, DB), lambda i: (0, i)),
            pl.BlockSpec((1, DB), lambda i: (0, i)),
        ],
        out_specs=pl.BlockSpec((128, DB), lambda i: (0, i)),
        out_shape=jax.ShapeDtypeStruct((128, 10240), f32),
    )(z128, p["d_ft_init"]["W"], b2(p["d_ft_init"]["b"]))
    zz = zzfull[:NB].reshape(N, LC)

    Ad1, Bd1 = pl.pallas_call(
        _dpre_body,
        grid=(GRID,),
        in_specs=[_rows((RB, 128)), _full(oha), _full(ohb),
                  _full(Wd1[:LC]), _full(Wd1[LC + NPG:2 * LC + NPG])],
        out_specs=[_rows((RB, 128))] * 2,
        out_shape=[jax.ShapeDtypeStruct((N, 128), f32)] * 2,
    )(zz, oha, ohb, Wd1[:LC], Wd1[LC + NPG:2 * LC + NPG])

    Sd1 = segsum(Bd1, srcg, dstg)
    d1, Ad2, Bd2 = combine(Ad1, Sd1, b2(p["d_conv1"]["b"]), Wd2[:LC], Wd2[LC:])
    Sd2 = segsum(Bd2, srcg, dstg)
    d2, Ad3, Bd3 = combine(Ad2, Sd2, b2(p["d_conv2"]["b"]), Wd3[:LC], Wd3[LC:])
    Sd3 = segsum(Bd3, srcg, dstg)
    d3 = combine(Ad3, Sd3, b2(p["d_conv3"]["b"]))

    o16 = pl.pallas_call(
        _heads_body,
        grid=(GRID,),
        in_specs=[_rows((RB, 128)), _full(Wcat), _full(bcat),
                  _full(W2blk), _full(b2cat), _full(We16)],
        out_specs=_rows((RB, 16)),
        out_shape=jax.ShapeDtypeStruct((N, 16), f32),
    )(d3, Wcat, bcat, W2blk, b2cat, We16)

    exist = o16[:, 11:12]
    pos_o = o16[:, 0:2]
    size_o = o16[:, 2:4]
    b_shape_o = o16[:, 4:10]
    b_iou_o = o16[:, 10:11]
    return (exist, pos_o, size_o, mu, log_var, b_shape_o, b_iou_o)
